# Initial kernel scaffold; baseline (speedup 1.0000x reference)
import jax
import jax.numpy as jnp
from jax.experimental import pallas as pl


def kernel(x, token_segment_indices, gate_w0, up_w0, down_w0, gate_w1, up_w1, down_w1, gate_w2, up_w2, down_w2, gate_w3, up_w3, down_w3):
    raise NotImplementedError("write your pallas kernel here")


# speedup vs baseline: 1.8613x; 1.8613x over previous
import functools

import jax
import jax.numpy as jnp
from jax import lax
from jax.experimental import pallas as pl
from jax.experimental.pallas import tpu as pltpu
from jax.experimental.pallas import tpu_sc as plsc

D_M = 2048
SEG = 2048
N_TOK = 8192
E = 4
T = 256
N_PAD = N_TOK + E * T
G = N_PAD // T

NC, NS = 2, 16
NW = NC * NS


@functools.cache
def _make_sc_gather(n_out: int, ch: int):
    rpw = n_out // NW
    n_chunks = rpw // ch
    assert rpw % ch == 0 and ch % 8 == 0 and rpw % 8 == 0
    mesh = plsc.VectorSubcoreMesh(core_axis_name="c", subcore_axis_name="s",
                                  num_cores=NC, num_subcores=NS)

    @functools.partial(
        pl.kernel,
        out_type=jax.ShapeDtypeStruct((n_out, D_M), jnp.float32),
        mesh=mesh,
        scratch_types=[
            pltpu.VMEM((rpw,), jnp.int32),
            pltpu.VMEM((ch, D_M), jnp.float32),
            pltpu.SemaphoreType.DMA,
        ],
    )
    def gather_k(table_hbm, idx_hbm, out_hbm, idx_v, buf, sem):
        w = lax.axis_index("s") * NC + lax.axis_index("c")
        base = w * rpw
        pltpu.sync_copy(idx_hbm.at[pl.ds(base, rpw)], idx_v)

        def body(c, carry):
            pltpu.async_copy(
                table_hbm.at[idx_v.at[pl.ds(c * ch, ch)]], buf, sem).wait()
            pltpu.sync_copy(buf, out_hbm.at[pl.ds(base + c * ch, ch)])
            return carry

        lax.fori_loop(0, n_chunks, body, 0, unroll=False)

    return gather_k


KC = 1024
DC = 1024
_DN = (((1,), (1,)), ((), ()))


def _hidden_body(be_ref, x_ref, g_ref, u_ref, h_ref):
    del be_ref
    x = x_ref[...]
    g = lax.dot_general(x, g_ref[0], _DN, preferred_element_type=jnp.float32)
    u = lax.dot_general(x, u_ref[0], _DN, preferred_element_type=jnp.float32)
    h_ref[...] = g * jax.nn.sigmoid(g) * u


def _down_body(be_ref, h_ref, d_ref, o_ref):
    del be_ref
    o_ref[...] = lax.dot_general(h_ref[...], d_ref[0], _DN,
                                 preferred_element_type=jnp.float32)


def _ffn_tc(block_expert, x_sorted, gws, uws, dws):
    hid_spec = pltpu.PrefetchScalarGridSpec(
        num_scalar_prefetch=1,
        grid=(SEG // KC, G),
        in_specs=[
            pl.BlockSpec((T, D_M), lambda k, b, be: (b, 0)),
            pl.BlockSpec((1, KC, D_M), lambda k, b, be: (be[b], k, 0)),
            pl.BlockSpec((1, KC, D_M), lambda k, b, be: (be[b], k, 0)),
        ],
        out_specs=pl.BlockSpec((T, KC), lambda k, b, be: (b, k)),
    )
    h = pl.pallas_call(
        _hidden_body,
        grid_spec=hid_spec,
        out_shape=jax.ShapeDtypeStruct((N_PAD, SEG), jnp.float32),
    )(block_expert, x_sorted, gws, uws)

    down_spec = pltpu.PrefetchScalarGridSpec(
        num_scalar_prefetch=1,
        grid=(D_M // DC, G),
        in_specs=[
            pl.BlockSpec((T, SEG), lambda k, b, be: (b, 0)),
            pl.BlockSpec((1, DC, SEG), lambda k, b, be: (be[b], k, 0)),
        ],
        out_specs=pl.BlockSpec((T, DC), lambda k, b, be: (b, k)),
    )
    return pl.pallas_call(
        _down_body,
        grid_spec=down_spec,
        out_shape=jax.ShapeDtypeStruct((N_PAD, D_M), jnp.float32),
    )(block_expert, h, dws)


def kernel(x, token_segment_indices, gate_w0, up_w0, down_w0, gate_w1,
           up_w1, down_w1, gate_w2, up_w2, down_w2, gate_w3, up_w3,
           down_w3):
    seg = token_segment_indices.astype(jnp.int32)

    oh = (seg[:, None] == jnp.arange(E, dtype=jnp.int32)[None, :])
    cum = jnp.cumsum(oh.astype(jnp.int32), axis=0)
    counts = cum[-1]
    rank = jnp.take_along_axis(cum, seg[:, None], axis=1)[:, 0] - 1
    pc = ((counts + T - 1) // T) * T
    po = jnp.concatenate(
        [jnp.zeros((1,), jnp.int32), jnp.cumsum(pc)[:E - 1].astype(jnp.int32)])
    pos = po[seg] + rank
    gather_ids = jnp.zeros((N_PAD,), jnp.int32).at[pos].set(
        jnp.arange(N_TOK, dtype=jnp.int32))
    block_expert = (jnp.searchsorted(
        po, jnp.arange(G, dtype=jnp.int32) * T, side="right") - 1
    ).astype(jnp.int32)

    gws = jnp.stack([gate_w0, gate_w1, gate_w2, gate_w3])
    uws = jnp.stack([up_w0, up_w1, up_w2, up_w3])
    dws = jnp.stack([down_w0, down_w1, down_w2, down_w3])

    x_sorted = _make_sc_gather(N_PAD, 24)(x, gather_ids)
    y_sorted = _ffn_tc(block_expert, x_sorted, gws, uws, dws)
    return _make_sc_gather(N_TOK, 16)(y_sorted, pos)


# speedup vs baseline: 1.8733x; 1.0065x over previous
import functools

import jax
import jax.numpy as jnp
from jax import lax
from jax.experimental import pallas as pl
from jax.experimental.pallas import tpu as pltpu
from jax.experimental.pallas import tpu_sc as plsc

D_M = 2048
SEG = 2048
N_TOK = 8192
E = 4
T = 256
N_PAD = N_TOK + E * T
G = N_PAD // T

NC, NS = 2, 16
NW = NC * NS


@functools.cache
def _make_sc_gather(n_out: int, ch: int):
    rpw = n_out // NW
    n_chunks = rpw // ch
    assert rpw % ch == 0 and ch % 8 == 0 and rpw % 8 == 0
    mesh = plsc.VectorSubcoreMesh(core_axis_name="c", subcore_axis_name="s",
                                  num_cores=NC, num_subcores=NS)

    @functools.partial(
        pl.kernel,
        out_type=jax.ShapeDtypeStruct((n_out, D_M), jnp.float32),
        mesh=mesh,
        scratch_types=[
            pltpu.VMEM((rpw,), jnp.int32),
            pltpu.VMEM((ch, D_M), jnp.float32),
            pltpu.SemaphoreType.DMA,
        ],
    )
    def gather_k(table_hbm, idx_hbm, out_hbm, idx_v, buf, sem):
        w = lax.axis_index("s") * NC + lax.axis_index("c")
        base = w * rpw
        pltpu.sync_copy(idx_hbm.at[pl.ds(base, rpw)], idx_v)

        def body(c, carry):
            pltpu.async_copy(
                table_hbm.at[idx_v.at[pl.ds(c * ch, ch)]], buf, sem).wait()
            pltpu.sync_copy(buf, out_hbm.at[pl.ds(base + c * ch, ch)])
            return carry

        lax.fori_loop(0, n_chunks, body, 0, unroll=False)

    return gather_k


KC = 1024
DC = 1024
_DN = (((1,), (1,)), ((), ()))


def _hidden_body(be_ref, x_ref, g_ref, u_ref, h_ref):
    del be_ref
    x = x_ref[...]
    g = lax.dot_general(x, g_ref[0], _DN, preferred_element_type=jnp.float32)
    u = lax.dot_general(x, u_ref[0], _DN, preferred_element_type=jnp.float32)
    h_ref[...] = g * jax.nn.sigmoid(g) * u


def _down_body(be_ref, h_ref, d_ref, o_ref):
    del be_ref
    o_ref[...] = lax.dot_general(h_ref[...], d_ref[0], _DN,
                                 preferred_element_type=jnp.float32)


def _ffn_tc(block_expert, x_sorted, gws, uws, dws):
    hid_spec = pltpu.PrefetchScalarGridSpec(
        num_scalar_prefetch=1,
        grid=(SEG // KC, G),
        in_specs=[
            pl.BlockSpec((T, D_M), lambda k, b, be: (b, 0)),
            pl.BlockSpec((1, KC, D_M), lambda k, b, be: (be[b], k, 0)),
            pl.BlockSpec((1, KC, D_M), lambda k, b, be: (be[b], k, 0)),
        ],
        out_specs=pl.BlockSpec((T, KC), lambda k, b, be: (b, k)),
    )
    h = pl.pallas_call(
        _hidden_body,
        grid_spec=hid_spec,
        out_shape=jax.ShapeDtypeStruct((N_PAD, SEG), jnp.float32),
    )(block_expert, x_sorted, gws, uws)

    down_spec = pltpu.PrefetchScalarGridSpec(
        num_scalar_prefetch=1,
        grid=(D_M // DC, G),
        in_specs=[
            pl.BlockSpec((T, SEG), lambda k, b, be: (b, 0)),
            pl.BlockSpec((1, DC, SEG), lambda k, b, be: (be[b], k, 0)),
        ],
        out_specs=pl.BlockSpec((T, DC), lambda k, b, be: (b, k)),
    )
    return pl.pallas_call(
        _down_body,
        grid_spec=down_spec,
        out_shape=jax.ShapeDtypeStruct((N_PAD, D_M), jnp.float32),
    )(block_expert, h, dws)


def kernel(x, token_segment_indices, gate_w0, up_w0, down_w0, gate_w1,
           up_w1, down_w1, gate_w2, up_w2, down_w2, gate_w3, up_w3,
           down_w3):
    seg = token_segment_indices.astype(jnp.int32)

    oh = (seg[:, None] == jnp.arange(E, dtype=jnp.int32)[None, :])
    cum = jnp.cumsum(oh.astype(jnp.int32), axis=0)
    counts = cum[-1]
    rank = jnp.take_along_axis(cum, seg[:, None], axis=1)[:, 0] - 1
    pc = ((counts + T - 1) // T) * T
    po = jnp.concatenate(
        [jnp.zeros((1,), jnp.int32), jnp.cumsum(pc)[:E - 1].astype(jnp.int32)])
    pos = po[seg] + rank
    gather_ids = jnp.zeros((N_PAD,), jnp.int32).at[pos].set(
        jnp.arange(N_TOK, dtype=jnp.int32))
    block_expert = (jnp.searchsorted(
        po, jnp.arange(G, dtype=jnp.int32) * T, side="right") - 1
    ).astype(jnp.int32)

    gws = jnp.stack([gate_w0, gate_w1, gate_w2, gate_w3])
    uws = jnp.stack([up_w0, up_w1, up_w2, up_w3])
    dws = jnp.stack([down_w0, down_w1, down_w2, down_w3])

    x_sorted = _make_sc_gather(N_PAD, 32)(x, gather_ids)
    y_sorted = _ffn_tc(block_expert, x_sorted, gws, uws, dws)
    return _make_sc_gather(N_TOK, 32)(y_sorted, pos)
